# baseline (speedup 1.0000x reference)
import jax
import jax.numpy as jnp
from jax.experimental import pallas as pl
from jax.experimental.pallas import tpu as pltpu

HIDDEN = 1024
EXPERT_INTER = 1024
NUM_EXPERTS = 16
VOCAB = 100000
N_TOKENS = 128
EPB = 2


def _moe_body(tid_ref, x_ref, gatew_ref, upw_ref, dnw_ref, out_ref):
    g = pl.program_id(0)

    @pl.when(g == 0)
    def _init():
        out_ref[...] = jnp.zeros_like(out_ref)

    x = x_ref[...].astype(jnp.bfloat16)
    tid = jnp.clip(tid_ref[...], 0, VOCAB - 1)
    eid = jax.lax.rem(tid, NUM_EXPERTS)

    acc = jnp.zeros_like(out_ref)
    for s in range(EPB):
        gate = jnp.dot(x, gatew_ref[s].astype(jnp.bfloat16),
                       preferred_element_type=jnp.float32)
        up = jnp.dot(x, upw_ref[s].astype(jnp.bfloat16),
                     preferred_element_type=jnp.float32)
        act = gate * jax.nn.sigmoid(gate) * up
        y = jnp.dot(act.astype(jnp.bfloat16), dnw_ref[s].astype(jnp.bfloat16),
                    preferred_element_type=jnp.float32)
        mask = eid == g * EPB + s
        acc = acc + jnp.where(mask, y, 0.0)
    out_ref[...] += acc


def kernel(x, token_ids, gate_up_proj, down_proj):
    n = x.shape[0]
    tid2d = token_ids.reshape(n, 1).astype(jnp.int32)
    return pl.pallas_call(
        _moe_body,
        grid=(NUM_EXPERTS // EPB,),
        in_specs=[
            pl.BlockSpec((n, 1), lambda g: (0, 0)),
            pl.BlockSpec((n, HIDDEN), lambda g: (0, 0)),
            pl.BlockSpec((EPB, HIDDEN, EXPERT_INTER), lambda g: (g, 0, 0)),
            pl.BlockSpec((EPB, HIDDEN, EXPERT_INTER), lambda g: (g, 0, 1)),
            pl.BlockSpec((EPB, EXPERT_INTER, HIDDEN), lambda g: (g, 0, 0)),
        ],
        out_specs=pl.BlockSpec((n, HIDDEN), lambda g: (0, 0)),
        out_shape=jax.ShapeDtypeStruct((n, HIDDEN), jnp.float32),
        compiler_params=pltpu.CompilerParams(
            dimension_semantics=("arbitrary",),
        ),
    )(tid2d, x, gate_up_proj, gate_up_proj, down_proj)
